# Initial kernel scaffold; baseline (speedup 1.0000x reference)
import jax
import jax.numpy as jnp
from jax.experimental import pallas as pl


def kernel(xyz1, xyz2, feat1, feat2, W0, b0, g0, beta0, W1, b1, g1, beta1):
    raise NotImplementedError("write your pallas kernel here")


# speedup vs baseline: 30.4858x; 30.4858x over previous
import jax
import jax.numpy as jnp
from jax.experimental import pallas as pl
from jax.experimental.pallas import tpu as pltpu

B, N1, N2 = 8, 4096, 1024
C1, C2 = 128, 256
NBLK = 512
NBLK_B = 2048


def _pass_a_body(xyz1_ref, xyz2t_ref, feat1_ref, feat2_ref, w0a_ref, w0b_ref,
                 b0_ref, y0_ref, s0_ref, f2_scr):
    b = pl.program_id(0)
    i = pl.program_id(1)

    @pl.when(i == 0)
    def _():
        f2_scr[...] = jax.lax.dot_general(
            w0b_ref[...], feat2_ref[0], (((1,), (0,)), ((), ())),
            preferred_element_type=jnp.float32)

    x1 = xyz1_ref[0]
    x2 = xyz2t_ref[0]
    dx = x1[:, 0:1] - x2[0:1, :]
    dy = x1[:, 1:2] - x2[1:2, :]
    dz = x1[:, 2:3] - x2[2:3, :]
    dist = (dx * dx + dy * dy) + dz * dz

    iota = jax.lax.broadcasted_iota(jnp.int32, (NBLK, N2), 1)
    d = dist
    ws = []
    idxs = []
    for _ in range(3):
        m = jnp.min(d, axis=1, keepdims=True)
        idx = jnp.min(jnp.where(d == m, iota, N2), axis=1, keepdims=True)
        ws.append(1.0 / (m + 1e-8))
        idxs.append(idx)
        d = jnp.where(iota == idx, jnp.float32(jnp.inf), d)
    wsum = ws[0] + ws[1] + ws[2]
    wmat = jnp.where(iota == idxs[0], ws[0] / wsum, 0.0)
    wmat = wmat + jnp.where(iota == idxs[1], ws[1] / wsum, 0.0)
    wmat = wmat + jnp.where(iota == idxs[2], ws[2] / wsum, 0.0)

    interp = jax.lax.dot_general(
        f2_scr[...], wmat, (((1,), (1,)), ((), ())),
        preferred_element_type=jnp.float32)
    y0 = interp + jax.lax.dot_general(
        w0a_ref[...], feat1_ref[0], (((1,), (0,)), ((), ())),
        preferred_element_type=jnp.float32) + b0_ref[...]
    y0_ref[0] = y0

    part = jnp.concatenate(
        [jnp.sum(y0, axis=1, keepdims=True),
         jnp.sum(y0 * y0, axis=1, keepdims=True)], axis=1)

    @pl.when((b == 0) & (i == 0))
    def _():
        s0_ref[...] = part

    @pl.when(~((b == 0) & (i == 0)))
    def _():
        s0_ref[...] = s0_ref[...] + part


def _pass_b_body(y0_ref, a0_ref, c0_ref, w1_ref, b1_ref, y1_ref, s1_ref):
    b = pl.program_id(0)
    i = pl.program_id(1)
    h0 = jnp.maximum(y0_ref[0] * a0_ref[...] + c0_ref[...], 0.0)
    y1 = jax.lax.dot_general(
        w1_ref[...], h0, (((1,), (0,)), ((), ())),
        preferred_element_type=jnp.float32) + b1_ref[...]
    y1_ref[0] = y1

    part = jnp.concatenate(
        [jnp.sum(y1, axis=1, keepdims=True),
         jnp.sum(y1 * y1, axis=1, keepdims=True)], axis=1)

    @pl.when((b == 0) & (i == 0))
    def _():
        s1_ref[...] = part

    @pl.when(~((b == 0) & (i == 0)))
    def _():
        s1_ref[...] = s1_ref[...] + part


def _pass_c_body(y1_ref, a1_ref, c1_ref, out_ref):
    out_ref[0] = jnp.maximum(y1_ref[0] * a1_ref[...] + c1_ref[...], 0.0)


def kernel(xyz1, xyz2, feat1, feat2, W0, b0, g0, beta0, W1, b1, g1, beta1):
    xyz2t = jnp.transpose(xyz2, (0, 2, 1))
    w0a = W0[:, :C1]
    w0b = W0[:, C1:]
    ntot = jnp.float32(B * N1)

    y0, s0 = pl.pallas_call(
        _pass_a_body,
        grid=(B, N1 // NBLK),
        in_specs=[
            pl.BlockSpec((1, NBLK, 3), lambda b, i: (b, i, 0)),
            pl.BlockSpec((1, 3, N2), lambda b, i: (b, 0, 0)),
            pl.BlockSpec((1, C1, NBLK), lambda b, i: (b, 0, i)),
            pl.BlockSpec((1, C2, N2), lambda b, i: (b, 0, 0)),
            pl.BlockSpec((C2, C1), lambda b, i: (0, 0)),
            pl.BlockSpec((C2, C2), lambda b, i: (0, 0)),
            pl.BlockSpec((C2, 1), lambda b, i: (0, 0)),
        ],
        out_specs=[
            pl.BlockSpec((1, C2, NBLK), lambda b, i: (b, 0, i)),
            pl.BlockSpec((C2, 2), lambda b, i: (0, 0)),
        ],
        out_shape=[
            jax.ShapeDtypeStruct((B, C2, N1), jnp.float32),
            jax.ShapeDtypeStruct((C2, 2), jnp.float32),
        ],
        scratch_shapes=[pltpu.VMEM((C2, N2), jnp.float32)],
    )(xyz1, xyz2t, feat1, feat2, w0a, w0b, b0.reshape(C2, 1))

    mean0 = s0[:, 0] / ntot
    var0 = s0[:, 1] / ntot - mean0 * mean0
    a0 = g0 / jnp.sqrt(var0 + 1e-5)
    c0 = beta0 - mean0 * a0

    y1, s1 = pl.pallas_call(
        _pass_b_body,
        grid=(B, N1 // NBLK_B),
        in_specs=[
            pl.BlockSpec((1, C2, NBLK_B), lambda b, i: (b, 0, i)),
            pl.BlockSpec((C2, 1), lambda b, i: (0, 0)),
            pl.BlockSpec((C2, 1), lambda b, i: (0, 0)),
            pl.BlockSpec((C1, C2), lambda b, i: (0, 0)),
            pl.BlockSpec((C1, 1), lambda b, i: (0, 0)),
        ],
        out_specs=[
            pl.BlockSpec((1, C1, NBLK_B), lambda b, i: (b, 0, i)),
            pl.BlockSpec((C1, 2), lambda b, i: (0, 0)),
        ],
        out_shape=[
            jax.ShapeDtypeStruct((B, C1, N1), jnp.float32),
            jax.ShapeDtypeStruct((C1, 2), jnp.float32),
        ],
    )(y0, a0.reshape(C2, 1), c0.reshape(C2, 1), W1, b1.reshape(C1, 1))

    mean1 = s1[:, 0] / ntot
    var1 = s1[:, 1] / ntot - mean1 * mean1
    a1 = g1 / jnp.sqrt(var1 + 1e-5)
    c1 = beta1 - mean1 * a1

    out = pl.pallas_call(
        _pass_c_body,
        grid=(B,),
        in_specs=[
            pl.BlockSpec((1, C1, N1), lambda b: (b, 0, 0)),
            pl.BlockSpec((C1, 1), lambda b: (0, 0)),
            pl.BlockSpec((C1, 1), lambda b: (0, 0)),
        ],
        out_specs=pl.BlockSpec((1, C1, N1), lambda b: (b, 0, 0)),
        out_shape=jax.ShapeDtypeStruct((B, C1, N1), jnp.float32),
    )(y1, a1.reshape(C1, 1), c1.reshape(C1, 1))

    return out
